# baseline (speedup 1.0000x reference)
import jax
import jax.numpy as jnp
from jax.experimental import pallas as pl
from jax.experimental.pallas import tpu as pltpu
from jax.experimental.pallas import tpu_sc as plsc

DIM = 1024
HID = 2048
ES = 8
EH = 2
K = 2
T = 2048

ROUTER_TILE = 512
TILE = 256
MAXT = 23
PADT = MAXT * TILE
NC, NS, L = 2, 16, 16
NW = NC * NS
TPW = T // NW
CT = 16


def _router_body(x_ref, g_ref, b_ref, swr_ref, sbr_ref, hwr_ref, hbr_ref,
                 xn_ref, sl_ref, sp_ref, ti_ref, tp_ref, hl_ref, hp_ref,
                 rank_ref, counts_ref, carry_ref):
    t = pl.program_id(0)

    @pl.when(t == 0)
    def _():
        carry_ref[...] = jnp.zeros_like(carry_ref)

    x = x_ref[...]
    mu = jnp.mean(x, axis=-1, keepdims=True)
    var = jnp.mean((x - mu) ** 2, axis=-1, keepdims=True)
    xn = (x - mu) / jnp.sqrt(var + 1e-5) * g_ref[0] + b_ref[0]
    xn_ref[...] = xn

    sl = jnp.dot(xn, swr_ref[...], preferred_element_type=jnp.float32) + sbr_ref[0]
    sl_ref[...] = sl
    sp = jax.nn.softmax(sl, axis=-1)
    sp_ref[...] = sp
    i1 = jnp.argmax(sp, axis=-1).astype(jnp.int32)
    v1 = jnp.max(sp, axis=-1)
    lane = jax.lax.broadcasted_iota(jnp.int32, sp.shape, 1)
    masked = jnp.where(lane == i1[:, None], -jnp.inf, sp)
    i2 = jnp.argmax(masked, axis=-1).astype(jnp.int32)
    v2 = jnp.max(masked, axis=-1)
    ti_ref[...] = jnp.stack([i1, i2], axis=-1)
    tp_ref[...] = jnp.stack([v1, v2], axis=-1)

    hl = jnp.dot(xn, hwr_ref[...], preferred_element_type=jnp.float32) + hbr_ref[0]
    hl_ref[...] = hl
    hp_ref[...] = jax.nn.softmax(hl, axis=-1)

    a1 = (lane == i1[:, None]).astype(jnp.float32)
    a2 = (lane == i2[:, None]).astype(jnp.float32)
    s = a1 + a2
    row = jax.lax.broadcasted_iota(jnp.int32, (ROUTER_TILE, ROUTER_TILE), 0)
    col = jax.lax.broadcasted_iota(jnp.int32, (ROUTER_TILE, ROUTER_TILE), 1)
    tri = (col < row).astype(jnp.float32)
    cum = jnp.dot(tri, s, preferred_element_type=jnp.float32) + carry_ref[...]
    r1 = jnp.sum(cum * a1, axis=-1)
    r2 = jnp.sum(cum * a2, axis=-1)
    rank_ref[...] = jnp.stack([r1, r2], axis=-1).astype(jnp.int32)
    carry = carry_ref[...] + jnp.sum(s, axis=0, keepdims=True)
    carry_ref[...] = carry
    counts_ref[...] = carry.astype(jnp.int32)


def _dispatch_body(counts_ref, ti_ref, rank_ref, tp_ref,
                   s0_ref, s1_ref, w0_ref, w1_ref, te_ref):
    c = counts_ref[...].astype(jnp.float32)
    pc = jnp.ceil(c / TILE) * TILE
    eidx = jax.lax.broadcasted_iota(jnp.int32, (ES, ES), 0)
    jidx = jax.lax.broadcasted_iota(jnp.int32, (ES, ES), 1)
    incl = (eidx <= jidx).astype(jnp.float32)
    cs = jnp.dot(pc, incl, preferred_element_type=jnp.float32)
    po = cs - pc

    ti = ti_ref[...]
    rank = rank_ref[...]
    lane = jax.lax.broadcasted_iota(jnp.int32, (T, ES), 1)
    po_b = jnp.broadcast_to(po, (T, ES))
    s0 = jnp.sum(jnp.where(lane == ti[:, 0:1], po_b, 0.0), axis=1)
    s1 = jnp.sum(jnp.where(lane == ti[:, 1:2], po_b, 0.0), axis=1)
    s0_ref[...] = (s0.astype(jnp.int32) + rank[:, 0])[:, None]
    s1_ref[...] = (s1.astype(jnp.int32) + rank[:, 1])[:, None]

    tp = tp_ref[...]
    w0_ref[...] = jnp.broadcast_to(tp[:, 0:1], (T, L))
    w1_ref[...] = jnp.broadcast_to(tp[:, 1:2], (T, L))

    tidx = jax.lax.broadcasted_iota(jnp.int32, (32, ES), 0).astype(jnp.float32)
    r16 = tidx * TILE
    cs_b = jnp.broadcast_to(cs, (32, ES))
    cnt = jnp.sum((r16 >= cs_b).astype(jnp.int32), axis=1)
    total = jnp.max(cs_b, axis=1) / TILE
    active = tidx[:, 0] < total
    te_ref[...] = jnp.where(active, cnt, -1)[:, None]


def _sc_scatter_body(xn_hbm, s0_hbm, s1_hbm, xg_hbm, rows_v, i0, i1, sem):
    wid = jax.lax.axis_index("s") * NC + jax.lax.axis_index("c")
    base = wid * TPW
    pltpu.sync_copy(xn_hbm.at[pl.ds(base, TPW)], rows_v)
    pltpu.sync_copy(s0_hbm.at[pl.ds(base, TPW)], i0)
    pltpu.sync_copy(s1_hbm.at[pl.ds(base, TPW)], i1)
    d0 = pltpu.async_copy(rows_v, xg_hbm.at[i0], sem)
    d1 = pltpu.async_copy(rows_v, xg_hbm.at[i1], sem)
    d0.wait()
    d1.wait()


SH_TILE = 1024


def _shared_body(xn_ref, hp_ref, w1_ref, b1_ref, w2_ref, b2_ref, out_ref):
    e = pl.program_id(0)
    tt = pl.program_id(1)
    xn = xn_ref[...]
    h = jnp.dot(xn, w1_ref[0], preferred_element_type=jnp.float32) + b1_ref[0]
    h = 0.5 * h * (1.0 + jax.lax.erf(h * 0.7071067811865476))
    y = jnp.dot(h, w2_ref[0], preferred_element_type=jnp.float32) + b2_ref[0]
    hp = hp_ref[...]
    lane = jax.lax.broadcasted_iota(jnp.int32, hp.shape, 1)
    w = jnp.sum(jnp.where(lane == e, hp, 0.0), axis=1, keepdims=True)
    rows = pl.ds(tt * SH_TILE, SH_TILE)

    @pl.when(e == 0)
    def _():
        out_ref[rows, :] = w * y

    @pl.when(e == 1)
    def _():
        out_ref[rows, :] += w * y


def _spec_body(te_ref, xg_ref, w1_ref, b1_ref, w2_ref, b2_ref, y_ref):
    i = pl.program_id(0)
    e = te_ref[i]

    @pl.when(e >= 0)
    def _():
        xg = xg_ref[...]
        h = jnp.dot(xg, w1_ref[0], preferred_element_type=jnp.float32) + b1_ref[0]
        h = 0.5 * h * (1.0 + jax.lax.erf(h * 0.7071067811865476))
        y_ref[...] = jnp.dot(h, w2_ref[0],
                             preferred_element_type=jnp.float32) + b2_ref[0]


def _sc_combine_body(ysp_hbm, sh_hbm, s0_hbm, s1_hbm, w0_hbm, w1_hbm, out_hbm,
                     r0a, r1a, sha, r0b, r1b, shb,
                     i0a, i1a, i0b, i1b, w0a, w1a, w0b, w1b,
                     gsa, gsb, osa, osb):
    wid = jax.lax.axis_index("s") * NC + jax.lax.axis_index("c")
    bufs = ((r0a, r1a, sha, i0a, i1a, w0a, w1a, gsa, osa),
            (r0b, r1b, shb, i0b, i1b, w0b, w1b, gsb, osb))

    def issue(c, p):
        r0, r1, sh, i0, i1, w0, w1, gs, _ = bufs[p]
        base = wid * TPW + c * CT
        pltpu.sync_copy(s0_hbm.at[pl.ds(base, CT)], i0)
        pltpu.sync_copy(s1_hbm.at[pl.ds(base, CT)], i1)
        pltpu.sync_copy(w0_hbm.at[pl.ds(base, CT), :], w0)
        pltpu.sync_copy(w1_hbm.at[pl.ds(base, CT), :], w1)
        return (pltpu.async_copy(ysp_hbm.at[i0], r0, gs),
                pltpu.async_copy(ysp_hbm.at[i1], r1, gs),
                pltpu.async_copy(sh_hbm.at[pl.ds(base, CT)], sh, gs))

    nchunk = TPW // CT
    gdesc = [None] * nchunk
    odesc = [None] * nchunk
    gdesc[0] = issue(0, 0)
    for c in range(nchunk):
        p = c % 2
        r0, r1, sh, _, _, w0, w1, _, osem = bufs[p]
        if c + 1 < nchunk:
            if c >= 1:
                odesc[c - 1].wait()
            gdesc[c + 1] = issue(c + 1, 1 - p)
        for d in gdesc[c]:
            d.wait()
        for t in range(CT):
            w0v = w0[t, :]
            w1v = w1[t, :]

            def body(d, carry, t=t, w0v=w0v, w1v=w1v, r0=r0, r1=r1, sh=sh):
                sl = pl.ds(d * L, L)
                sh[t, sl] = sh[t, sl] + w0v * r0[t, sl] + w1v * r1[t, sl]
                return carry

            jax.lax.fori_loop(0, DIM // L, body, 0)
        base = wid * TPW + c * CT
        odesc[c] = pltpu.async_copy(sh, out_hbm.at[pl.ds(base, CT)], osem)
    odesc[nchunk - 2].wait()
    odesc[nchunk - 1].wait()


def kernel(x, ln_g, ln_b, spec_Wr, spec_br, spec_W1, spec_b1, spec_W2, spec_b2,
           sh_Wr, sh_br, sh_W1, sh_b1, sh_W2, sh_b2):
    B = x.shape[0]
    x2 = x.reshape(T, DIM)

    n_rt = T // ROUTER_TILE
    router = pl.pallas_call(
        _router_body,
        grid=(n_rt,),
        in_specs=[
            pl.BlockSpec((ROUTER_TILE, DIM), lambda t: (t, 0)),
            pl.BlockSpec((1, DIM), lambda t: (0, 0)),
            pl.BlockSpec((1, DIM), lambda t: (0, 0)),
            pl.BlockSpec((DIM, ES), lambda t: (0, 0)),
            pl.BlockSpec((1, ES), lambda t: (0, 0)),
            pl.BlockSpec((DIM, EH), lambda t: (0, 0)),
            pl.BlockSpec((1, EH), lambda t: (0, 0)),
        ],
        out_specs=[
            pl.BlockSpec((ROUTER_TILE, DIM), lambda t: (t, 0)),
            pl.BlockSpec((ROUTER_TILE, ES), lambda t: (t, 0)),
            pl.BlockSpec((ROUTER_TILE, ES), lambda t: (t, 0)),
            pl.BlockSpec((ROUTER_TILE, K), lambda t: (t, 0)),
            pl.BlockSpec((ROUTER_TILE, K), lambda t: (t, 0)),
            pl.BlockSpec((ROUTER_TILE, EH), lambda t: (t, 0)),
            pl.BlockSpec((ROUTER_TILE, EH), lambda t: (t, 0)),
            pl.BlockSpec((ROUTER_TILE, K), lambda t: (t, 0)),
            pl.BlockSpec((1, ES), lambda t: (0, 0)),
        ],
        out_shape=[
            jax.ShapeDtypeStruct((T, DIM), jnp.float32),
            jax.ShapeDtypeStruct((T, ES), jnp.float32),
            jax.ShapeDtypeStruct((T, ES), jnp.float32),
            jax.ShapeDtypeStruct((T, K), jnp.int32),
            jax.ShapeDtypeStruct((T, K), jnp.float32),
            jax.ShapeDtypeStruct((T, EH), jnp.float32),
            jax.ShapeDtypeStruct((T, EH), jnp.float32),
            jax.ShapeDtypeStruct((T, K), jnp.int32),
            jax.ShapeDtypeStruct((1, ES), jnp.int32),
        ],
        scratch_shapes=[pltpu.VMEM((1, ES), jnp.float32)],
    )
    xn, sl, sp, ti, tp, hl, hp, rank, counts = router(
        x2, ln_g.reshape(1, DIM), ln_b.reshape(1, DIM),
        spec_Wr, spec_br.reshape(1, ES), sh_Wr, sh_br.reshape(1, EH))

    dispatch = pl.pallas_call(
        _dispatch_body,
        grid=(1,),
        in_specs=[
            pl.BlockSpec((1, ES), lambda i: (0, 0)),
            pl.BlockSpec((T, K), lambda i: (0, 0)),
            pl.BlockSpec((T, K), lambda i: (0, 0)),
            pl.BlockSpec((T, K), lambda i: (0, 0)),
        ],
        out_specs=[
            pl.BlockSpec((T, 1), lambda i: (0, 0)),
            pl.BlockSpec((T, 1), lambda i: (0, 0)),
            pl.BlockSpec((T, L), lambda i: (0, 0)),
            pl.BlockSpec((T, L), lambda i: (0, 0)),
            pl.BlockSpec((32, 1), lambda i: (0, 0)),
        ],
        out_shape=[
            jax.ShapeDtypeStruct((T, 1), jnp.int32),
            jax.ShapeDtypeStruct((T, 1), jnp.int32),
            jax.ShapeDtypeStruct((T, L), jnp.float32),
            jax.ShapeDtypeStruct((T, L), jnp.float32),
            jax.ShapeDtypeStruct((32, 1), jnp.int32),
        ],
    )
    s0, s1, w0e, w1e, te = dispatch(counts, ti, rank, tp)
    s0 = s0.reshape(T)
    s1 = s1.reshape(T)
    te = te.reshape(32)

    mesh = plsc.VectorSubcoreMesh(core_axis_name="c", subcore_axis_name="s",
                                  num_cores=NC, num_subcores=NS)
    sc_scatter = pl.kernel(
        _sc_scatter_body,
        out_type=jax.ShapeDtypeStruct((PADT, DIM), jnp.float32),
        mesh=mesh,
        scratch_types=[
            pltpu.VMEM((TPW, DIM), jnp.float32),
            pltpu.VMEM((TPW,), jnp.int32),
            pltpu.VMEM((TPW,), jnp.int32),
            pltpu.SemaphoreType.DMA,
        ],
    )
    xg = sc_scatter(xn, s0, s1)

    shared = pl.pallas_call(
        _shared_body,
        grid=(EH, T // SH_TILE),
        in_specs=[
            pl.BlockSpec((SH_TILE, DIM), lambda e, tt: (tt, 0)),
            pl.BlockSpec((SH_TILE, EH), lambda e, tt: (tt, 0)),
            pl.BlockSpec((1, DIM, HID), lambda e, tt: (e, 0, 0)),
            pl.BlockSpec((1, 1, HID), lambda e, tt: (e, 0, 0)),
            pl.BlockSpec((1, HID, DIM), lambda e, tt: (e, 0, 0)),
            pl.BlockSpec((1, 1, DIM), lambda e, tt: (e, 0, 0)),
        ],
        out_specs=pl.BlockSpec((T, DIM), lambda e, tt: (0, 0)),
        out_shape=jax.ShapeDtypeStruct((T, DIM), jnp.float32),
    )
    sh_out = shared(xn, hp, sh_W1, sh_b1.reshape(EH, 1, HID),
                    sh_W2, sh_b2.reshape(EH, 1, DIM))

    spec = pl.pallas_call(
        _spec_body,
        grid_spec=pltpu.PrefetchScalarGridSpec(
            num_scalar_prefetch=1,
            grid=(MAXT,),
            in_specs=[
                pl.BlockSpec((TILE, DIM), lambda i, te_r: (i, 0)),
                pl.BlockSpec((1, DIM, HID),
                             lambda i, te_r: (jnp.clip(te_r[i], 0, ES - 1), 0, 0)),
                pl.BlockSpec((1, 1, HID),
                             lambda i, te_r: (jnp.clip(te_r[i], 0, ES - 1), 0, 0)),
                pl.BlockSpec((1, HID, DIM),
                             lambda i, te_r: (jnp.clip(te_r[i], 0, ES - 1), 0, 0)),
                pl.BlockSpec((1, 1, DIM),
                             lambda i, te_r: (jnp.clip(te_r[i], 0, ES - 1), 0, 0)),
            ],
            out_specs=pl.BlockSpec((TILE, DIM), lambda i, te_r: (i, 0)),
        ),
        out_shape=jax.ShapeDtypeStruct((PADT, DIM), jnp.float32),
    )
    ysp = spec(te, xg,
               spec_W1, spec_b1.reshape(ES, 1, HID),
               spec_W2, spec_b2.reshape(ES, 1, DIM))

    sc_combine = pl.kernel(
        _sc_combine_body,
        out_type=jax.ShapeDtypeStruct((T, DIM), jnp.float32),
        mesh=mesh,
        scratch_types=[
            pltpu.VMEM((CT, DIM), jnp.float32),
            pltpu.VMEM((CT, DIM), jnp.float32),
            pltpu.VMEM((CT, DIM), jnp.float32),
            pltpu.VMEM((CT, DIM), jnp.float32),
            pltpu.VMEM((CT, DIM), jnp.float32),
            pltpu.VMEM((CT, DIM), jnp.float32),
            pltpu.VMEM((CT,), jnp.int32),
            pltpu.VMEM((CT,), jnp.int32),
            pltpu.VMEM((CT,), jnp.int32),
            pltpu.VMEM((CT,), jnp.int32),
            pltpu.VMEM((CT, L), jnp.float32),
            pltpu.VMEM((CT, L), jnp.float32),
            pltpu.VMEM((CT, L), jnp.float32),
            pltpu.VMEM((CT, L), jnp.float32),
            pltpu.SemaphoreType.DMA,
            pltpu.SemaphoreType.DMA,
            pltpu.SemaphoreType.DMA,
            pltpu.SemaphoreType.DMA,
        ],
    )
    out = sc_combine(ysp, sh_out, s0, s1, w0e, w1e)

    return (out.reshape(B, T, DIM), sl.reshape(B, T, ES), sp.reshape(B, T, ES),
            ti.reshape(B, T, K), tp.reshape(B, T, K), hl.reshape(B, T, EH),
            hp.reshape(B, T, EH))


# speedup vs baseline: 1.0683x; 1.0683x over previous
import jax
import jax.numpy as jnp
from jax.experimental import pallas as pl
from jax.experimental.pallas import tpu as pltpu
from jax.experimental.pallas import tpu_sc as plsc

DIM = 1024
HID = 2048
ES = 8
EH = 2
K = 2
T = 2048

ROUTER_TILE = 512
TILE = 512
MAXT = 15
PADT = MAXT * TILE
NC, NS, L = 2, 16, 16
NW = NC * NS
TPW = T // NW
CT = 16


def _router_body(x_ref, g_ref, b_ref, swr_ref, sbr_ref, hwr_ref, hbr_ref,
                 xn_ref, sl_ref, sp_ref, ti_ref, tp_ref, hl_ref, hp_ref,
                 rank_ref, counts_ref, carry_ref):
    t = pl.program_id(0)

    @pl.when(t == 0)
    def _():
        carry_ref[...] = jnp.zeros_like(carry_ref)

    x = x_ref[...]
    mu = jnp.mean(x, axis=-1, keepdims=True)
    var = jnp.mean((x - mu) ** 2, axis=-1, keepdims=True)
    xn = (x - mu) / jnp.sqrt(var + 1e-5) * g_ref[0] + b_ref[0]
    xn_ref[...] = xn

    sl = jnp.dot(xn, swr_ref[...], preferred_element_type=jnp.float32) + sbr_ref[0]
    sl_ref[...] = sl
    sp = jax.nn.softmax(sl, axis=-1)
    sp_ref[...] = sp
    i1 = jnp.argmax(sp, axis=-1).astype(jnp.int32)
    v1 = jnp.max(sp, axis=-1)
    lane = jax.lax.broadcasted_iota(jnp.int32, sp.shape, 1)
    masked = jnp.where(lane == i1[:, None], -jnp.inf, sp)
    i2 = jnp.argmax(masked, axis=-1).astype(jnp.int32)
    v2 = jnp.max(masked, axis=-1)
    ti_ref[...] = jnp.stack([i1, i2], axis=-1)
    tp_ref[...] = jnp.stack([v1, v2], axis=-1)

    hl = jnp.dot(xn, hwr_ref[...], preferred_element_type=jnp.float32) + hbr_ref[0]
    hl_ref[...] = hl
    hp_ref[...] = jax.nn.softmax(hl, axis=-1)

    a1 = (lane == i1[:, None]).astype(jnp.float32)
    a2 = (lane == i2[:, None]).astype(jnp.float32)
    s = a1 + a2
    row = jax.lax.broadcasted_iota(jnp.int32, (ROUTER_TILE, ROUTER_TILE), 0)
    col = jax.lax.broadcasted_iota(jnp.int32, (ROUTER_TILE, ROUTER_TILE), 1)
    tri = (col < row).astype(jnp.float32)
    cum = jnp.dot(tri, s, preferred_element_type=jnp.float32) + carry_ref[...]
    r1 = jnp.sum(cum * a1, axis=-1)
    r2 = jnp.sum(cum * a2, axis=-1)
    rank_ref[...] = jnp.stack([r1, r2], axis=-1).astype(jnp.int32)
    carry = carry_ref[...] + jnp.sum(s, axis=0, keepdims=True)
    carry_ref[...] = carry
    counts_ref[...] = carry.astype(jnp.int32)


def _dispatch_body(counts_ref, ti_ref, rank_ref, tp_ref,
                   s0_ref, s1_ref, w0_ref, w1_ref, te_ref):
    c = counts_ref[...].astype(jnp.float32)
    pc = jnp.ceil(c / TILE) * TILE
    eidx = jax.lax.broadcasted_iota(jnp.int32, (ES, ES), 0)
    jidx = jax.lax.broadcasted_iota(jnp.int32, (ES, ES), 1)
    incl = (eidx <= jidx).astype(jnp.float32)
    cs = jnp.dot(pc, incl, preferred_element_type=jnp.float32)
    po = cs - pc

    ti = ti_ref[...]
    rank = rank_ref[...]
    lane = jax.lax.broadcasted_iota(jnp.int32, (T, ES), 1)
    po_b = jnp.broadcast_to(po, (T, ES))
    s0 = jnp.sum(jnp.where(lane == ti[:, 0:1], po_b, 0.0), axis=1)
    s1 = jnp.sum(jnp.where(lane == ti[:, 1:2], po_b, 0.0), axis=1)
    s0_ref[...] = (s0.astype(jnp.int32) + rank[:, 0])[:, None]
    s1_ref[...] = (s1.astype(jnp.int32) + rank[:, 1])[:, None]

    tp = tp_ref[...]
    w0_ref[...] = jnp.broadcast_to(tp[:, 0:1], (T, L))
    w1_ref[...] = jnp.broadcast_to(tp[:, 1:2], (T, L))

    tidx = jax.lax.broadcasted_iota(jnp.int32, (32, ES), 0).astype(jnp.float32)
    r16 = tidx * TILE
    cs_b = jnp.broadcast_to(cs, (32, ES))
    cnt = jnp.sum((r16 >= cs_b).astype(jnp.int32), axis=1)
    total = jnp.max(cs_b, axis=1) / TILE
    active = tidx[:, 0] < total
    te_ref[...] = jnp.where(active, cnt, -1)[:, None]


def _sc_scatter_body(xn_hbm, s0_hbm, s1_hbm, xg_hbm, rows_v, i0, i1, sem):
    wid = jax.lax.axis_index("s") * NC + jax.lax.axis_index("c")
    base = wid * TPW
    pltpu.sync_copy(xn_hbm.at[pl.ds(base, TPW)], rows_v)
    pltpu.sync_copy(s0_hbm.at[pl.ds(base, TPW)], i0)
    pltpu.sync_copy(s1_hbm.at[pl.ds(base, TPW)], i1)
    d0 = pltpu.async_copy(rows_v, xg_hbm.at[i0], sem)
    d1 = pltpu.async_copy(rows_v, xg_hbm.at[i1], sem)
    d0.wait()
    d1.wait()


SH_TILE = 1024


def _shared_body(xn_ref, hp_ref, w1_ref, b1_ref, w2_ref, b2_ref, out_ref):
    e = pl.program_id(0)
    tt = pl.program_id(1)
    xn = xn_ref[...]
    h = jnp.dot(xn, w1_ref[0], preferred_element_type=jnp.float32) + b1_ref[0]
    h = 0.5 * h * (1.0 + jax.lax.erf(h * 0.7071067811865476))
    y = jnp.dot(h, w2_ref[0], preferred_element_type=jnp.float32) + b2_ref[0]
    hp = hp_ref[...]
    lane = jax.lax.broadcasted_iota(jnp.int32, hp.shape, 1)
    w = jnp.sum(jnp.where(lane == e, hp, 0.0), axis=1, keepdims=True)
    rows = pl.ds(tt * SH_TILE, SH_TILE)

    @pl.when(e == 0)
    def _():
        out_ref[rows, :] = w * y

    @pl.when(e == 1)
    def _():
        out_ref[rows, :] += w * y


def _spec_body(te_ref, xg_ref, w1_ref, b1_ref, w2_ref, b2_ref, y_ref):
    i = pl.program_id(0)
    e = te_ref[i]

    @pl.when(e >= 0)
    def _():
        xg = xg_ref[...]
        h = jnp.dot(xg, w1_ref[0], preferred_element_type=jnp.float32) + b1_ref[0]
        h = 0.5 * h * (1.0 + jax.lax.erf(h * 0.7071067811865476))
        y_ref[...] = jnp.dot(h, w2_ref[0],
                             preferred_element_type=jnp.float32) + b2_ref[0]


def _sc_combine_body(ysp_hbm, sh_hbm, s0_hbm, s1_hbm, w0_hbm, w1_hbm, out_hbm,
                     r0a, r1a, sha, r0b, r1b, shb,
                     i0a, i1a, i0b, i1b, w0a, w1a, w0b, w1b,
                     gsa, gsb, osa, osb):
    wid = jax.lax.axis_index("s") * NC + jax.lax.axis_index("c")
    bufs = ((r0a, r1a, sha, i0a, i1a, w0a, w1a, gsa, osa),
            (r0b, r1b, shb, i0b, i1b, w0b, w1b, gsb, osb))

    def issue(c, p):
        r0, r1, sh, i0, i1, w0, w1, gs, _ = bufs[p]
        base = wid * TPW + c * CT
        pltpu.sync_copy(s0_hbm.at[pl.ds(base, CT)], i0)
        pltpu.sync_copy(s1_hbm.at[pl.ds(base, CT)], i1)
        pltpu.sync_copy(w0_hbm.at[pl.ds(base, CT), :], w0)
        pltpu.sync_copy(w1_hbm.at[pl.ds(base, CT), :], w1)
        return (pltpu.async_copy(ysp_hbm.at[i0], r0, gs),
                pltpu.async_copy(ysp_hbm.at[i1], r1, gs),
                pltpu.async_copy(sh_hbm.at[pl.ds(base, CT)], sh, gs))

    nchunk = TPW // CT
    gdesc = [None] * nchunk
    odesc = [None] * nchunk
    gdesc[0] = issue(0, 0)
    for c in range(nchunk):
        p = c % 2
        r0, r1, sh, _, _, w0, w1, _, osem = bufs[p]
        if c + 1 < nchunk:
            if c >= 1:
                odesc[c - 1].wait()
            gdesc[c + 1] = issue(c + 1, 1 - p)
        for d in gdesc[c]:
            d.wait()
        for t in range(CT):
            w0v = w0[t, :]
            w1v = w1[t, :]

            def body(d, carry, t=t, w0v=w0v, w1v=w1v, r0=r0, r1=r1, sh=sh):
                for u in range(4):
                    sl = pl.ds((d * 4 + u) * L, L)
                    sh[t, sl] = sh[t, sl] + w0v * r0[t, sl] + w1v * r1[t, sl]
                return carry

            jax.lax.fori_loop(0, DIM // (4 * L), body, 0)
        base = wid * TPW + c * CT
        odesc[c] = pltpu.async_copy(sh, out_hbm.at[pl.ds(base, CT)], osem)
    odesc[nchunk - 2].wait()
    odesc[nchunk - 1].wait()


def kernel(x, ln_g, ln_b, spec_Wr, spec_br, spec_W1, spec_b1, spec_W2, spec_b2,
           sh_Wr, sh_br, sh_W1, sh_b1, sh_W2, sh_b2):
    B = x.shape[0]
    x2 = x.reshape(T, DIM)

    n_rt = T // ROUTER_TILE
    router = pl.pallas_call(
        _router_body,
        grid=(n_rt,),
        in_specs=[
            pl.BlockSpec((ROUTER_TILE, DIM), lambda t: (t, 0)),
            pl.BlockSpec((1, DIM), lambda t: (0, 0)),
            pl.BlockSpec((1, DIM), lambda t: (0, 0)),
            pl.BlockSpec((DIM, ES), lambda t: (0, 0)),
            pl.BlockSpec((1, ES), lambda t: (0, 0)),
            pl.BlockSpec((DIM, EH), lambda t: (0, 0)),
            pl.BlockSpec((1, EH), lambda t: (0, 0)),
        ],
        out_specs=[
            pl.BlockSpec((ROUTER_TILE, DIM), lambda t: (t, 0)),
            pl.BlockSpec((ROUTER_TILE, ES), lambda t: (t, 0)),
            pl.BlockSpec((ROUTER_TILE, ES), lambda t: (t, 0)),
            pl.BlockSpec((ROUTER_TILE, K), lambda t: (t, 0)),
            pl.BlockSpec((ROUTER_TILE, K), lambda t: (t, 0)),
            pl.BlockSpec((ROUTER_TILE, EH), lambda t: (t, 0)),
            pl.BlockSpec((ROUTER_TILE, EH), lambda t: (t, 0)),
            pl.BlockSpec((ROUTER_TILE, K), lambda t: (t, 0)),
            pl.BlockSpec((1, ES), lambda t: (0, 0)),
        ],
        out_shape=[
            jax.ShapeDtypeStruct((T, DIM), jnp.float32),
            jax.ShapeDtypeStruct((T, ES), jnp.float32),
            jax.ShapeDtypeStruct((T, ES), jnp.float32),
            jax.ShapeDtypeStruct((T, K), jnp.int32),
            jax.ShapeDtypeStruct((T, K), jnp.float32),
            jax.ShapeDtypeStruct((T, EH), jnp.float32),
            jax.ShapeDtypeStruct((T, EH), jnp.float32),
            jax.ShapeDtypeStruct((T, K), jnp.int32),
            jax.ShapeDtypeStruct((1, ES), jnp.int32),
        ],
        scratch_shapes=[pltpu.VMEM((1, ES), jnp.float32)],
    )
    xn, sl, sp, ti, tp, hl, hp, rank, counts = router(
        x2, ln_g.reshape(1, DIM), ln_b.reshape(1, DIM),
        spec_Wr, spec_br.reshape(1, ES), sh_Wr, sh_br.reshape(1, EH))

    dispatch = pl.pallas_call(
        _dispatch_body,
        grid=(1,),
        in_specs=[
            pl.BlockSpec((1, ES), lambda i: (0, 0)),
            pl.BlockSpec((T, K), lambda i: (0, 0)),
            pl.BlockSpec((T, K), lambda i: (0, 0)),
            pl.BlockSpec((T, K), lambda i: (0, 0)),
        ],
        out_specs=[
            pl.BlockSpec((T, 1), lambda i: (0, 0)),
            pl.BlockSpec((T, 1), lambda i: (0, 0)),
            pl.BlockSpec((T, L), lambda i: (0, 0)),
            pl.BlockSpec((T, L), lambda i: (0, 0)),
            pl.BlockSpec((32, 1), lambda i: (0, 0)),
        ],
        out_shape=[
            jax.ShapeDtypeStruct((T, 1), jnp.int32),
            jax.ShapeDtypeStruct((T, 1), jnp.int32),
            jax.ShapeDtypeStruct((T, L), jnp.float32),
            jax.ShapeDtypeStruct((T, L), jnp.float32),
            jax.ShapeDtypeStruct((32, 1), jnp.int32),
        ],
    )
    s0, s1, w0e, w1e, te = dispatch(counts, ti, rank, tp)
    s0 = s0.reshape(T)
    s1 = s1.reshape(T)
    te = te.reshape(32)

    mesh = plsc.VectorSubcoreMesh(core_axis_name="c", subcore_axis_name="s",
                                  num_cores=NC, num_subcores=NS)
    sc_scatter = pl.kernel(
        _sc_scatter_body,
        out_type=jax.ShapeDtypeStruct((PADT, DIM), jnp.float32),
        mesh=mesh,
        scratch_types=[
            pltpu.VMEM((TPW, DIM), jnp.float32),
            pltpu.VMEM((TPW,), jnp.int32),
            pltpu.VMEM((TPW,), jnp.int32),
            pltpu.SemaphoreType.DMA,
        ],
    )
    xg = sc_scatter(xn, s0, s1)

    shared = pl.pallas_call(
        _shared_body,
        grid=(EH, T // SH_TILE),
        in_specs=[
            pl.BlockSpec((SH_TILE, DIM), lambda e, tt: (tt, 0)),
            pl.BlockSpec((SH_TILE, EH), lambda e, tt: (tt, 0)),
            pl.BlockSpec((1, DIM, HID), lambda e, tt: (e, 0, 0)),
            pl.BlockSpec((1, 1, HID), lambda e, tt: (e, 0, 0)),
            pl.BlockSpec((1, HID, DIM), lambda e, tt: (e, 0, 0)),
            pl.BlockSpec((1, 1, DIM), lambda e, tt: (e, 0, 0)),
        ],
        out_specs=pl.BlockSpec((T, DIM), lambda e, tt: (0, 0)),
        out_shape=jax.ShapeDtypeStruct((T, DIM), jnp.float32),
    )
    sh_out = shared(xn, hp, sh_W1, sh_b1.reshape(EH, 1, HID),
                    sh_W2, sh_b2.reshape(EH, 1, DIM))

    spec = pl.pallas_call(
        _spec_body,
        grid_spec=pltpu.PrefetchScalarGridSpec(
            num_scalar_prefetch=1,
            grid=(MAXT,),
            in_specs=[
                pl.BlockSpec((TILE, DIM), lambda i, te_r: (i, 0)),
                pl.BlockSpec((1, DIM, HID),
                             lambda i, te_r: (jnp.clip(te_r[i], 0, ES - 1), 0, 0)),
                pl.BlockSpec((1, 1, HID),
                             lambda i, te_r: (jnp.clip(te_r[i], 0, ES - 1), 0, 0)),
                pl.BlockSpec((1, HID, DIM),
                             lambda i, te_r: (jnp.clip(te_r[i], 0, ES - 1), 0, 0)),
                pl.BlockSpec((1, 1, DIM),
                             lambda i, te_r: (jnp.clip(te_r[i], 0, ES - 1), 0, 0)),
            ],
            out_specs=pl.BlockSpec((TILE, DIM), lambda i, te_r: (i, 0)),
        ),
        out_shape=jax.ShapeDtypeStruct((PADT, DIM), jnp.float32),
    )
    ysp = spec(te, xg,
               spec_W1, spec_b1.reshape(ES, 1, HID),
               spec_W2, spec_b2.reshape(ES, 1, DIM))

    sc_combine = pl.kernel(
        _sc_combine_body,
        out_type=jax.ShapeDtypeStruct((T, DIM), jnp.float32),
        mesh=mesh,
        scratch_types=[
            pltpu.VMEM((CT, DIM), jnp.float32),
            pltpu.VMEM((CT, DIM), jnp.float32),
            pltpu.VMEM((CT, DIM), jnp.float32),
            pltpu.VMEM((CT, DIM), jnp.float32),
            pltpu.VMEM((CT, DIM), jnp.float32),
            pltpu.VMEM((CT, DIM), jnp.float32),
            pltpu.VMEM((CT,), jnp.int32),
            pltpu.VMEM((CT,), jnp.int32),
            pltpu.VMEM((CT,), jnp.int32),
            pltpu.VMEM((CT,), jnp.int32),
            pltpu.VMEM((CT, L), jnp.float32),
            pltpu.VMEM((CT, L), jnp.float32),
            pltpu.VMEM((CT, L), jnp.float32),
            pltpu.VMEM((CT, L), jnp.float32),
            pltpu.SemaphoreType.DMA,
            pltpu.SemaphoreType.DMA,
            pltpu.SemaphoreType.DMA,
            pltpu.SemaphoreType.DMA,
        ],
    )
    out = sc_combine(ysp, sh_out, s0, s1, w0e, w1e)

    return (out.reshape(B, T, DIM), sl.reshape(B, T, ES), sp.reshape(B, T, ES),
            ti.reshape(B, T, K), tp.reshape(B, T, K), hl.reshape(B, T, EH),
            hp.reshape(B, T, EH))


# speedup vs baseline: 1.0779x; 1.0089x over previous
import jax
import jax.numpy as jnp
from jax.experimental import pallas as pl
from jax.experimental.pallas import tpu as pltpu
from jax.experimental.pallas import tpu_sc as plsc

DIM = 1024
HID = 2048
ES = 8
EH = 2
K = 2
T = 2048

ROUTER_TILE = 512
TILE = 512
MAXT = 15
PADT = MAXT * TILE
NC, NS, L = 2, 16, 16
NW = NC * NS
TPW = T // NW
CT = 16


def _router_body(x_ref, g_ref, b_ref, swr_ref, sbr_ref, hwr_ref, hbr_ref,
                 xn_ref, sl_ref, sp_ref, ti_ref, tp_ref, hl_ref, hp_ref,
                 rank_ref, counts_ref, carry_ref):
    t = pl.program_id(0)

    @pl.when(t == 0)
    def _():
        carry_ref[...] = jnp.zeros_like(carry_ref)

    x = x_ref[...]
    mu = jnp.mean(x, axis=-1, keepdims=True)
    var = jnp.mean((x - mu) ** 2, axis=-1, keepdims=True)
    xn = (x - mu) / jnp.sqrt(var + 1e-5) * g_ref[0] + b_ref[0]
    xn_ref[...] = xn

    sl = jnp.dot(xn, swr_ref[...], preferred_element_type=jnp.float32) + sbr_ref[0]
    sl_ref[...] = sl
    sp = jax.nn.softmax(sl, axis=-1)
    sp_ref[...] = sp
    i1 = jnp.argmax(sp, axis=-1).astype(jnp.int32)
    v1 = jnp.max(sp, axis=-1)
    lane = jax.lax.broadcasted_iota(jnp.int32, sp.shape, 1)
    masked = jnp.where(lane == i1[:, None], -jnp.inf, sp)
    i2 = jnp.argmax(masked, axis=-1).astype(jnp.int32)
    v2 = jnp.max(masked, axis=-1)
    ti_ref[...] = jnp.stack([i1, i2], axis=-1)
    tp_ref[...] = jnp.stack([v1, v2], axis=-1)

    hl = jnp.dot(xn, hwr_ref[...], preferred_element_type=jnp.float32) + hbr_ref[0]
    hl_ref[...] = hl
    hp_ref[...] = jax.nn.softmax(hl, axis=-1)

    a1 = (lane == i1[:, None]).astype(jnp.float32)
    a2 = (lane == i2[:, None]).astype(jnp.float32)
    s = a1 + a2
    row = jax.lax.broadcasted_iota(jnp.int32, (ROUTER_TILE, ROUTER_TILE), 0)
    col = jax.lax.broadcasted_iota(jnp.int32, (ROUTER_TILE, ROUTER_TILE), 1)
    tri = (col < row).astype(jnp.float32)
    cum = jnp.dot(tri, s, preferred_element_type=jnp.float32) + carry_ref[...]
    r1 = jnp.sum(cum * a1, axis=-1)
    r2 = jnp.sum(cum * a2, axis=-1)
    rank_ref[...] = jnp.stack([r1, r2], axis=-1).astype(jnp.int32)
    carry = carry_ref[...] + jnp.sum(s, axis=0, keepdims=True)
    carry_ref[...] = carry
    counts_ref[...] = carry.astype(jnp.int32)


def _dispatch_body(counts_ref, ti_ref, rank_ref, tp_ref,
                   s0_ref, s1_ref, w0_ref, w1_ref, te_ref):
    c = counts_ref[...].astype(jnp.float32)
    pc = jnp.ceil(c / TILE) * TILE
    eidx = jax.lax.broadcasted_iota(jnp.int32, (ES, ES), 0)
    jidx = jax.lax.broadcasted_iota(jnp.int32, (ES, ES), 1)
    incl = (eidx <= jidx).astype(jnp.float32)
    cs = jnp.dot(pc, incl, preferred_element_type=jnp.float32)
    po = cs - pc

    ti = ti_ref[...]
    rank = rank_ref[...]
    lane = jax.lax.broadcasted_iota(jnp.int32, (T, ES), 1)
    po_b = jnp.broadcast_to(po, (T, ES))
    s0 = jnp.sum(jnp.where(lane == ti[:, 0:1], po_b, 0.0), axis=1)
    s1 = jnp.sum(jnp.where(lane == ti[:, 1:2], po_b, 0.0), axis=1)
    s0_ref[...] = (s0.astype(jnp.int32) + rank[:, 0])[:, None]
    s1_ref[...] = (s1.astype(jnp.int32) + rank[:, 1])[:, None]

    tp = tp_ref[...]
    w0_ref[...] = jnp.broadcast_to(tp[:, 0:1], (T, L))
    w1_ref[...] = jnp.broadcast_to(tp[:, 1:2], (T, L))

    tidx = jax.lax.broadcasted_iota(jnp.int32, (32, ES), 0).astype(jnp.float32)
    r16 = tidx * TILE
    cs_b = jnp.broadcast_to(cs, (32, ES))
    cnt = jnp.sum((r16 >= cs_b).astype(jnp.int32), axis=1)
    total = jnp.max(cs_b, axis=1) / TILE
    active = tidx[:, 0] < total
    te_ref[...] = jnp.where(active, cnt, -1)[:, None]


def _sc_scatter_body(xn_hbm, s0_hbm, s1_hbm, xg_hbm, rows_v, i0, i1, sem):
    wid = jax.lax.axis_index("s") * NC + jax.lax.axis_index("c")
    base = wid * TPW
    pltpu.sync_copy(xn_hbm.at[pl.ds(base, TPW)], rows_v)
    pltpu.sync_copy(s0_hbm.at[pl.ds(base, TPW)], i0)
    pltpu.sync_copy(s1_hbm.at[pl.ds(base, TPW)], i1)
    d0 = pltpu.async_copy(rows_v, xg_hbm.at[i0], sem)
    d1 = pltpu.async_copy(rows_v, xg_hbm.at[i1], sem)
    d0.wait()
    d1.wait()


SH_TILE = 1024


def _shared_body(xn_ref, hp_ref, w1_ref, b1_ref, w2_ref, b2_ref, out_ref):
    e = pl.program_id(0)
    tt = pl.program_id(1)
    xn = xn_ref[...]
    h = jnp.dot(xn, w1_ref[0], preferred_element_type=jnp.float32) + b1_ref[0]
    h = 0.5 * h * (1.0 + jax.lax.erf(h * 0.7071067811865476))
    y = jnp.dot(h, w2_ref[0], preferred_element_type=jnp.float32) + b2_ref[0]
    hp = hp_ref[...]
    lane = jax.lax.broadcasted_iota(jnp.int32, hp.shape, 1)
    w = jnp.sum(jnp.where(lane == e, hp, 0.0), axis=1, keepdims=True)
    rows = pl.ds(tt * SH_TILE, SH_TILE)

    @pl.when(e == 0)
    def _():
        out_ref[rows, :] = w * y

    @pl.when(e == 1)
    def _():
        out_ref[rows, :] += w * y


def _spec_body(te_ref, xg_ref, w1_ref, b1_ref, w2_ref, b2_ref, y_ref):
    i = pl.program_id(0)
    e = te_ref[i]

    @pl.when(e >= 0)
    def _():
        xg = xg_ref[...]
        h = jnp.dot(xg, w1_ref[0], preferred_element_type=jnp.float32) + b1_ref[0]
        h = 0.5 * h * (1.0 + jax.lax.erf(h * 0.7071067811865476))
        y_ref[...] = jnp.dot(h, w2_ref[0],
                             preferred_element_type=jnp.float32) + b2_ref[0]


def _sc_combine_body(ysp_hbm, sh_hbm, s0_hbm, s1_hbm, w0_hbm, w1_hbm, out_hbm,
                     r0a, r1a, sha, r0b, r1b, shb,
                     i0all, i1all, w0all, w1all,
                     gsa, gsb, osa, osb):
    wid = jax.lax.axis_index("s") * NC + jax.lax.axis_index("c")
    base0 = wid * TPW
    pltpu.sync_copy(s0_hbm.at[pl.ds(base0, TPW)], i0all)
    pltpu.sync_copy(s1_hbm.at[pl.ds(base0, TPW)], i1all)
    pltpu.sync_copy(w0_hbm.at[pl.ds(base0, TPW), :], w0all)
    pltpu.sync_copy(w1_hbm.at[pl.ds(base0, TPW), :], w1all)
    bufs = ((r0a, r1a, sha, gsa, osa),
            (r0b, r1b, shb, gsb, osb))

    def issue(c, p):
        r0, r1, sh, gs, _ = bufs[p]
        lo = c * CT
        return (pltpu.async_copy(ysp_hbm.at[i0all.at[pl.ds(lo, CT)]], r0, gs),
                pltpu.async_copy(ysp_hbm.at[i1all.at[pl.ds(lo, CT)]], r1, gs),
                pltpu.async_copy(sh_hbm.at[pl.ds(base0 + lo, CT)], sh, gs))

    nchunk = TPW // CT
    gdesc = [None] * nchunk
    odesc = [None] * nchunk
    gdesc[0] = issue(0, 0)
    for c in range(nchunk):
        p = c % 2
        r0, r1, sh, _, osem = bufs[p]
        if c + 1 < nchunk:
            if c >= 1:
                odesc[c - 1].wait()
            gdesc[c + 1] = issue(c + 1, 1 - p)
        for d in gdesc[c]:
            d.wait()
        for t in range(CT):
            w0v = w0all[c * CT + t, :]
            w1v = w1all[c * CT + t, :]

            def body(d, carry, t=t, w0v=w0v, w1v=w1v, r0=r0, r1=r1, sh=sh):
                for u in range(4):
                    sl = pl.ds((d * 4 + u) * L, L)
                    sh[t, sl] = sh[t, sl] + w0v * r0[t, sl] + w1v * r1[t, sl]
                return carry

            jax.lax.fori_loop(0, DIM // (4 * L), body, 0)
        odesc[c] = pltpu.async_copy(sh, out_hbm.at[pl.ds(base0 + c * CT, CT)],
                                    osem)
    odesc[nchunk - 2].wait()
    odesc[nchunk - 1].wait()


def kernel(x, ln_g, ln_b, spec_Wr, spec_br, spec_W1, spec_b1, spec_W2, spec_b2,
           sh_Wr, sh_br, sh_W1, sh_b1, sh_W2, sh_b2):
    B = x.shape[0]
    x2 = x.reshape(T, DIM)

    n_rt = T // ROUTER_TILE
    router = pl.pallas_call(
        _router_body,
        grid=(n_rt,),
        in_specs=[
            pl.BlockSpec((ROUTER_TILE, DIM), lambda t: (t, 0)),
            pl.BlockSpec((1, DIM), lambda t: (0, 0)),
            pl.BlockSpec((1, DIM), lambda t: (0, 0)),
            pl.BlockSpec((DIM, ES), lambda t: (0, 0)),
            pl.BlockSpec((1, ES), lambda t: (0, 0)),
            pl.BlockSpec((DIM, EH), lambda t: (0, 0)),
            pl.BlockSpec((1, EH), lambda t: (0, 0)),
        ],
        out_specs=[
            pl.BlockSpec((ROUTER_TILE, DIM), lambda t: (t, 0)),
            pl.BlockSpec((ROUTER_TILE, ES), lambda t: (t, 0)),
            pl.BlockSpec((ROUTER_TILE, ES), lambda t: (t, 0)),
            pl.BlockSpec((ROUTER_TILE, K), lambda t: (t, 0)),
            pl.BlockSpec((ROUTER_TILE, K), lambda t: (t, 0)),
            pl.BlockSpec((ROUTER_TILE, EH), lambda t: (t, 0)),
            pl.BlockSpec((ROUTER_TILE, EH), lambda t: (t, 0)),
            pl.BlockSpec((ROUTER_TILE, K), lambda t: (t, 0)),
            pl.BlockSpec((1, ES), lambda t: (0, 0)),
        ],
        out_shape=[
            jax.ShapeDtypeStruct((T, DIM), jnp.float32),
            jax.ShapeDtypeStruct((T, ES), jnp.float32),
            jax.ShapeDtypeStruct((T, ES), jnp.float32),
            jax.ShapeDtypeStruct((T, K), jnp.int32),
            jax.ShapeDtypeStruct((T, K), jnp.float32),
            jax.ShapeDtypeStruct((T, EH), jnp.float32),
            jax.ShapeDtypeStruct((T, EH), jnp.float32),
            jax.ShapeDtypeStruct((T, K), jnp.int32),
            jax.ShapeDtypeStruct((1, ES), jnp.int32),
        ],
        scratch_shapes=[pltpu.VMEM((1, ES), jnp.float32)],
    )
    xn, sl, sp, ti, tp, hl, hp, rank, counts = router(
        x2, ln_g.reshape(1, DIM), ln_b.reshape(1, DIM),
        spec_Wr, spec_br.reshape(1, ES), sh_Wr, sh_br.reshape(1, EH))

    dispatch = pl.pallas_call(
        _dispatch_body,
        grid=(1,),
        in_specs=[
            pl.BlockSpec((1, ES), lambda i: (0, 0)),
            pl.BlockSpec((T, K), lambda i: (0, 0)),
            pl.BlockSpec((T, K), lambda i: (0, 0)),
            pl.BlockSpec((T, K), lambda i: (0, 0)),
        ],
        out_specs=[
            pl.BlockSpec((T, 1), lambda i: (0, 0)),
            pl.BlockSpec((T, 1), lambda i: (0, 0)),
            pl.BlockSpec((T, L), lambda i: (0, 0)),
            pl.BlockSpec((T, L), lambda i: (0, 0)),
            pl.BlockSpec((32, 1), lambda i: (0, 0)),
        ],
        out_shape=[
            jax.ShapeDtypeStruct((T, 1), jnp.int32),
            jax.ShapeDtypeStruct((T, 1), jnp.int32),
            jax.ShapeDtypeStruct((T, L), jnp.float32),
            jax.ShapeDtypeStruct((T, L), jnp.float32),
            jax.ShapeDtypeStruct((32, 1), jnp.int32),
        ],
    )
    s0, s1, w0e, w1e, te = dispatch(counts, ti, rank, tp)
    s0 = s0.reshape(T)
    s1 = s1.reshape(T)
    te = te.reshape(32)

    mesh = plsc.VectorSubcoreMesh(core_axis_name="c", subcore_axis_name="s",
                                  num_cores=NC, num_subcores=NS)
    sc_scatter = pl.kernel(
        _sc_scatter_body,
        out_type=jax.ShapeDtypeStruct((PADT, DIM), jnp.float32),
        mesh=mesh,
        scratch_types=[
            pltpu.VMEM((TPW, DIM), jnp.float32),
            pltpu.VMEM((TPW,), jnp.int32),
            pltpu.VMEM((TPW,), jnp.int32),
            pltpu.SemaphoreType.DMA,
        ],
    )
    xg = sc_scatter(xn, s0, s1)

    shared = pl.pallas_call(
        _shared_body,
        grid=(EH, T // SH_TILE),
        in_specs=[
            pl.BlockSpec((SH_TILE, DIM), lambda e, tt: (tt, 0)),
            pl.BlockSpec((SH_TILE, EH), lambda e, tt: (tt, 0)),
            pl.BlockSpec((1, DIM, HID), lambda e, tt: (e, 0, 0)),
            pl.BlockSpec((1, 1, HID), lambda e, tt: (e, 0, 0)),
            pl.BlockSpec((1, HID, DIM), lambda e, tt: (e, 0, 0)),
            pl.BlockSpec((1, 1, DIM), lambda e, tt: (e, 0, 0)),
        ],
        out_specs=pl.BlockSpec((T, DIM), lambda e, tt: (0, 0)),
        out_shape=jax.ShapeDtypeStruct((T, DIM), jnp.float32),
    )
    sh_out = shared(xn, hp, sh_W1, sh_b1.reshape(EH, 1, HID),
                    sh_W2, sh_b2.reshape(EH, 1, DIM))

    spec = pl.pallas_call(
        _spec_body,
        grid_spec=pltpu.PrefetchScalarGridSpec(
            num_scalar_prefetch=1,
            grid=(MAXT,),
            in_specs=[
                pl.BlockSpec((TILE, DIM), lambda i, te_r: (i, 0)),
                pl.BlockSpec((1, DIM, HID),
                             lambda i, te_r: (jnp.clip(te_r[i], 0, ES - 1), 0, 0)),
                pl.BlockSpec((1, 1, HID),
                             lambda i, te_r: (jnp.clip(te_r[i], 0, ES - 1), 0, 0)),
                pl.BlockSpec((1, HID, DIM),
                             lambda i, te_r: (jnp.clip(te_r[i], 0, ES - 1), 0, 0)),
                pl.BlockSpec((1, 1, DIM),
                             lambda i, te_r: (jnp.clip(te_r[i], 0, ES - 1), 0, 0)),
            ],
            out_specs=pl.BlockSpec((TILE, DIM), lambda i, te_r: (i, 0)),
        ),
        out_shape=jax.ShapeDtypeStruct((PADT, DIM), jnp.float32),
    )
    ysp = spec(te, xg,
               spec_W1, spec_b1.reshape(ES, 1, HID),
               spec_W2, spec_b2.reshape(ES, 1, DIM))

    sc_combine = pl.kernel(
        _sc_combine_body,
        out_type=jax.ShapeDtypeStruct((T, DIM), jnp.float32),
        mesh=mesh,
        scratch_types=[
            pltpu.VMEM((CT, DIM), jnp.float32),
            pltpu.VMEM((CT, DIM), jnp.float32),
            pltpu.VMEM((CT, DIM), jnp.float32),
            pltpu.VMEM((CT, DIM), jnp.float32),
            pltpu.VMEM((CT, DIM), jnp.float32),
            pltpu.VMEM((CT, DIM), jnp.float32),
            pltpu.VMEM((TPW,), jnp.int32),
            pltpu.VMEM((TPW,), jnp.int32),
            pltpu.VMEM((TPW, L), jnp.float32),
            pltpu.VMEM((TPW, L), jnp.float32),
            pltpu.SemaphoreType.DMA,
            pltpu.SemaphoreType.DMA,
            pltpu.SemaphoreType.DMA,
            pltpu.SemaphoreType.DMA,
        ],
    )
    out = sc_combine(ysp, sh_out, s0, s1, w0e, w1e)

    return (out.reshape(B, T, DIM), sl.reshape(B, T, ES), sp.reshape(B, T, ES),
            ti.reshape(B, T, K), tp.reshape(B, T, K), hl.reshape(B, T, EH),
            hp.reshape(B, T, EH))


# speedup vs baseline: 1.0833x; 1.0051x over previous
import jax
import jax.numpy as jnp
from jax.experimental import pallas as pl
from jax.experimental.pallas import tpu as pltpu
from jax.experimental.pallas import tpu_sc as plsc

DIM = 1024
HID = 2048
ES = 8
EH = 2
K = 2
T = 2048

ROUTER_TILE = 512
TILE = 512
MAXT = 15
PADT = MAXT * TILE
NC, NS, L = 2, 16, 16
NW = NC * NS
TPW = T // NW
CT = 16


def _router_body(x_ref, g_ref, b_ref, swr_ref, sbr_ref, hwr_ref, hbr_ref,
                 xn_ref, sl_ref, sp_ref, ti_ref, tp_ref, hl_ref, hp_ref,
                 s0_ref, s1_ref, w0_ref, w1_ref, te_ref,
                 carry_ref, ti_scr, rank_scr, tp_scr):
    t = pl.program_id(0)
    rows = pl.ds(t * ROUTER_TILE, ROUTER_TILE)

    @pl.when(t == 0)
    def _():
        carry_ref[...] = jnp.zeros_like(carry_ref)

    x = x_ref[...]
    mu = jnp.mean(x, axis=-1, keepdims=True)
    var = jnp.mean((x - mu) ** 2, axis=-1, keepdims=True)
    xn = (x - mu) / jnp.sqrt(var + 1e-5) * g_ref[0] + b_ref[0]
    xn_ref[...] = xn

    sl = jnp.dot(xn, swr_ref[...], preferred_element_type=jnp.float32) + sbr_ref[0]
    sl_ref[...] = sl
    sp = jax.nn.softmax(sl, axis=-1)
    sp_ref[...] = sp
    i1 = jnp.argmax(sp, axis=-1).astype(jnp.int32)
    v1 = jnp.max(sp, axis=-1)
    lane = jax.lax.broadcasted_iota(jnp.int32, sp.shape, 1)
    masked = jnp.where(lane == i1[:, None], -jnp.inf, sp)
    i2 = jnp.argmax(masked, axis=-1).astype(jnp.int32)
    v2 = jnp.max(masked, axis=-1)
    tiv = jnp.stack([i1, i2], axis=-1)
    tpv = jnp.stack([v1, v2], axis=-1)
    ti_ref[...] = tiv
    tp_ref[...] = tpv
    ti_scr[rows, :] = tiv
    tp_scr[rows, :] = tpv

    hl = jnp.dot(xn, hwr_ref[...], preferred_element_type=jnp.float32) + hbr_ref[0]
    hl_ref[...] = hl
    hp_ref[...] = jax.nn.softmax(hl, axis=-1)

    a1 = (lane == i1[:, None]).astype(jnp.float32)
    a2 = (lane == i2[:, None]).astype(jnp.float32)
    s = a1 + a2
    row = jax.lax.broadcasted_iota(jnp.int32, (ROUTER_TILE, ROUTER_TILE), 0)
    col = jax.lax.broadcasted_iota(jnp.int32, (ROUTER_TILE, ROUTER_TILE), 1)
    tri = (col < row).astype(jnp.float32)
    cum = jnp.dot(tri, s, preferred_element_type=jnp.float32) + carry_ref[...]
    r1 = jnp.sum(cum * a1, axis=-1)
    r2 = jnp.sum(cum * a2, axis=-1)
    rank_scr[rows, :] = jnp.stack([r1, r2], axis=-1).astype(jnp.int32)
    carry = carry_ref[...] + jnp.sum(s, axis=0, keepdims=True)
    carry_ref[...] = carry

    @pl.when(t == pl.num_programs(0) - 1)
    def _():
        c = carry
        pc = jnp.ceil(c / TILE) * TILE
        eidx = jax.lax.broadcasted_iota(jnp.int32, (ES, ES), 0)
        jidx = jax.lax.broadcasted_iota(jnp.int32, (ES, ES), 1)
        incl = (eidx <= jidx).astype(jnp.float32)
        cs = jnp.dot(pc, incl, preferred_element_type=jnp.float32)
        po = cs - pc

        ti_a = ti_scr[...]
        rank_a = rank_scr[...]
        lane_t = jax.lax.broadcasted_iota(jnp.int32, (T, ES), 1)
        po_b = jnp.broadcast_to(po, (T, ES))
        s0 = jnp.sum(jnp.where(lane_t == ti_a[:, 0:1], po_b, 0.0), axis=1)
        s1 = jnp.sum(jnp.where(lane_t == ti_a[:, 1:2], po_b, 0.0), axis=1)
        s0_ref[...] = (s0.astype(jnp.int32) + rank_a[:, 0])[:, None]
        s1_ref[...] = (s1.astype(jnp.int32) + rank_a[:, 1])[:, None]

        tp_a = tp_scr[...]
        w0_ref[...] = jnp.broadcast_to(tp_a[:, 0:1], (T, L))
        w1_ref[...] = jnp.broadcast_to(tp_a[:, 1:2], (T, L))

        tidx = jax.lax.broadcasted_iota(jnp.int32, (32, ES), 0).astype(jnp.float32)
        r32 = tidx * TILE
        cs_b = jnp.broadcast_to(cs, (32, ES))
        cnt = jnp.sum((r32 >= cs_b).astype(jnp.int32), axis=1)
        total = jnp.max(cs_b, axis=1) / TILE
        active = tidx[:, 0] < total
        te_ref[...] = jnp.where(active, cnt, -1)[:, None]


def _sc_scatter_body(xn_hbm, s0_hbm, s1_hbm, xg_hbm, rows_v, i0, i1, sem):
    wid = jax.lax.axis_index("s") * NC + jax.lax.axis_index("c")
    base = wid * TPW
    pltpu.sync_copy(xn_hbm.at[pl.ds(base, TPW)], rows_v)
    pltpu.sync_copy(s0_hbm.at[pl.ds(base, TPW)], i0)
    pltpu.sync_copy(s1_hbm.at[pl.ds(base, TPW)], i1)
    d0 = pltpu.async_copy(rows_v, xg_hbm.at[i0], sem)
    d1 = pltpu.async_copy(rows_v, xg_hbm.at[i1], sem)
    d0.wait()
    d1.wait()


SH_TILE = 1024


def _shared_body(xn_ref, hp_ref, w1_ref, b1_ref, w2_ref, b2_ref, out_ref):
    e = pl.program_id(0)
    tt = pl.program_id(1)
    xn = xn_ref[...]
    h = jnp.dot(xn, w1_ref[0], preferred_element_type=jnp.float32) + b1_ref[0]
    h = 0.5 * h * (1.0 + jax.lax.erf(h * 0.7071067811865476))
    y = jnp.dot(h, w2_ref[0], preferred_element_type=jnp.float32) + b2_ref[0]
    hp = hp_ref[...]
    lane = jax.lax.broadcasted_iota(jnp.int32, hp.shape, 1)
    w = jnp.sum(jnp.where(lane == e, hp, 0.0), axis=1, keepdims=True)
    rows = pl.ds(tt * SH_TILE, SH_TILE)

    @pl.when(e == 0)
    def _():
        out_ref[rows, :] = w * y

    @pl.when(e == 1)
    def _():
        out_ref[rows, :] += w * y


def _spec_body(te_ref, xg_ref, w1_ref, b1_ref, w2_ref, b2_ref, y_ref):
    i = pl.program_id(0)
    e = te_ref[i]

    @pl.when(e >= 0)
    def _():
        xg = xg_ref[...]
        h = jnp.dot(xg, w1_ref[0], preferred_element_type=jnp.float32) + b1_ref[0]
        h = 0.5 * h * (1.0 + jax.lax.erf(h * 0.7071067811865476))
        y_ref[...] = jnp.dot(h, w2_ref[0],
                             preferred_element_type=jnp.float32) + b2_ref[0]


def _sc_combine_body(ysp_hbm, sh_hbm, s0_hbm, s1_hbm, w0_hbm, w1_hbm, out_hbm,
                     r0a, r1a, sha, r0b, r1b, shb,
                     i0all, i1all, w0all, w1all,
                     gsa, gsb, osa, osb):
    wid = jax.lax.axis_index("s") * NC + jax.lax.axis_index("c")
    base0 = wid * TPW
    pltpu.sync_copy(s0_hbm.at[pl.ds(base0, TPW)], i0all)
    pltpu.sync_copy(s1_hbm.at[pl.ds(base0, TPW)], i1all)
    pltpu.sync_copy(w0_hbm.at[pl.ds(base0, TPW), :], w0all)
    pltpu.sync_copy(w1_hbm.at[pl.ds(base0, TPW), :], w1all)
    bufs = ((r0a, r1a, sha, gsa, osa),
            (r0b, r1b, shb, gsb, osb))

    def issue(c, p):
        r0, r1, sh, gs, _ = bufs[p]
        lo = c * CT
        return (pltpu.async_copy(ysp_hbm.at[i0all.at[pl.ds(lo, CT)]], r0, gs),
                pltpu.async_copy(ysp_hbm.at[i1all.at[pl.ds(lo, CT)]], r1, gs),
                pltpu.async_copy(sh_hbm.at[pl.ds(base0 + lo, CT)], sh, gs))

    nchunk = TPW // CT
    gdesc = [None] * nchunk
    odesc = [None] * nchunk
    gdesc[0] = issue(0, 0)
    for c in range(nchunk):
        p = c % 2
        r0, r1, sh, _, osem = bufs[p]
        if c + 1 < nchunk:
            if c >= 1:
                odesc[c - 1].wait()
            gdesc[c + 1] = issue(c + 1, 1 - p)
        for d in gdesc[c]:
            d.wait()
        for t in range(CT):
            w0v = w0all[c * CT + t, :]
            w1v = w1all[c * CT + t, :]

            def body(d, carry, t=t, w0v=w0v, w1v=w1v, r0=r0, r1=r1, sh=sh):
                for u in range(4):
                    sl = pl.ds((d * 4 + u) * L, L)
                    sh[t, sl] = sh[t, sl] + w0v * r0[t, sl] + w1v * r1[t, sl]
                return carry

            jax.lax.fori_loop(0, DIM // (4 * L), body, 0)
        odesc[c] = pltpu.async_copy(sh, out_hbm.at[pl.ds(base0 + c * CT, CT)],
                                    osem)
    odesc[nchunk - 2].wait()
    odesc[nchunk - 1].wait()


def kernel(x, ln_g, ln_b, spec_Wr, spec_br, spec_W1, spec_b1, spec_W2, spec_b2,
           sh_Wr, sh_br, sh_W1, sh_b1, sh_W2, sh_b2):
    B = x.shape[0]
    x2 = x.reshape(T, DIM)

    n_rt = T // ROUTER_TILE
    router = pl.pallas_call(
        _router_body,
        grid=(n_rt,),
        in_specs=[
            pl.BlockSpec((ROUTER_TILE, DIM), lambda t: (t, 0)),
            pl.BlockSpec((1, DIM), lambda t: (0, 0)),
            pl.BlockSpec((1, DIM), lambda t: (0, 0)),
            pl.BlockSpec((DIM, ES), lambda t: (0, 0)),
            pl.BlockSpec((1, ES), lambda t: (0, 0)),
            pl.BlockSpec((DIM, EH), lambda t: (0, 0)),
            pl.BlockSpec((1, EH), lambda t: (0, 0)),
        ],
        out_specs=[
            pl.BlockSpec((ROUTER_TILE, DIM), lambda t: (t, 0)),
            pl.BlockSpec((ROUTER_TILE, ES), lambda t: (t, 0)),
            pl.BlockSpec((ROUTER_TILE, ES), lambda t: (t, 0)),
            pl.BlockSpec((ROUTER_TILE, K), lambda t: (t, 0)),
            pl.BlockSpec((ROUTER_TILE, K), lambda t: (t, 0)),
            pl.BlockSpec((ROUTER_TILE, EH), lambda t: (t, 0)),
            pl.BlockSpec((ROUTER_TILE, EH), lambda t: (t, 0)),
            pl.BlockSpec((T, 1), lambda t: (0, 0)),
            pl.BlockSpec((T, 1), lambda t: (0, 0)),
            pl.BlockSpec((T, L), lambda t: (0, 0)),
            pl.BlockSpec((T, L), lambda t: (0, 0)),
            pl.BlockSpec((32, 1), lambda t: (0, 0)),
        ],
        out_shape=[
            jax.ShapeDtypeStruct((T, DIM), jnp.float32),
            jax.ShapeDtypeStruct((T, ES), jnp.float32),
            jax.ShapeDtypeStruct((T, ES), jnp.float32),
            jax.ShapeDtypeStruct((T, K), jnp.int32),
            jax.ShapeDtypeStruct((T, K), jnp.float32),
            jax.ShapeDtypeStruct((T, EH), jnp.float32),
            jax.ShapeDtypeStruct((T, EH), jnp.float32),
            jax.ShapeDtypeStruct((T, 1), jnp.int32),
            jax.ShapeDtypeStruct((T, 1), jnp.int32),
            jax.ShapeDtypeStruct((T, L), jnp.float32),
            jax.ShapeDtypeStruct((T, L), jnp.float32),
            jax.ShapeDtypeStruct((32, 1), jnp.int32),
        ],
        scratch_shapes=[
            pltpu.VMEM((1, ES), jnp.float32),
            pltpu.VMEM((T, K), jnp.int32),
            pltpu.VMEM((T, K), jnp.int32),
            pltpu.VMEM((T, K), jnp.float32),
        ],
    )
    (xn, sl, sp, ti, tp, hl, hp,
     s0, s1, w0e, w1e, te) = router(
        x2, ln_g.reshape(1, DIM), ln_b.reshape(1, DIM),
        spec_Wr, spec_br.reshape(1, ES), sh_Wr, sh_br.reshape(1, EH))

    s0 = s0.reshape(T)
    s1 = s1.reshape(T)
    te = te.reshape(32)

    mesh = plsc.VectorSubcoreMesh(core_axis_name="c", subcore_axis_name="s",
                                  num_cores=NC, num_subcores=NS)
    sc_scatter = pl.kernel(
        _sc_scatter_body,
        out_type=jax.ShapeDtypeStruct((PADT, DIM), jnp.float32),
        mesh=mesh,
        scratch_types=[
            pltpu.VMEM((TPW, DIM), jnp.float32),
            pltpu.VMEM((TPW,), jnp.int32),
            pltpu.VMEM((TPW,), jnp.int32),
            pltpu.SemaphoreType.DMA,
        ],
    )
    xg = sc_scatter(xn, s0, s1)

    shared = pl.pallas_call(
        _shared_body,
        grid=(EH, T // SH_TILE),
        in_specs=[
            pl.BlockSpec((SH_TILE, DIM), lambda e, tt: (tt, 0)),
            pl.BlockSpec((SH_TILE, EH), lambda e, tt: (tt, 0)),
            pl.BlockSpec((1, DIM, HID), lambda e, tt: (e, 0, 0)),
            pl.BlockSpec((1, 1, HID), lambda e, tt: (e, 0, 0)),
            pl.BlockSpec((1, HID, DIM), lambda e, tt: (e, 0, 0)),
            pl.BlockSpec((1, 1, DIM), lambda e, tt: (e, 0, 0)),
        ],
        out_specs=pl.BlockSpec((T, DIM), lambda e, tt: (0, 0)),
        out_shape=jax.ShapeDtypeStruct((T, DIM), jnp.float32),
    )
    sh_out = shared(xn, hp, sh_W1, sh_b1.reshape(EH, 1, HID),
                    sh_W2, sh_b2.reshape(EH, 1, DIM))

    spec = pl.pallas_call(
        _spec_body,
        grid_spec=pltpu.PrefetchScalarGridSpec(
            num_scalar_prefetch=1,
            grid=(MAXT,),
            in_specs=[
                pl.BlockSpec((TILE, DIM), lambda i, te_r: (i, 0)),
                pl.BlockSpec((1, DIM, HID),
                             lambda i, te_r: (jnp.clip(te_r[i], 0, ES - 1), 0, 0)),
                pl.BlockSpec((1, 1, HID),
                             lambda i, te_r: (jnp.clip(te_r[i], 0, ES - 1), 0, 0)),
                pl.BlockSpec((1, HID, DIM),
                             lambda i, te_r: (jnp.clip(te_r[i], 0, ES - 1), 0, 0)),
                pl.BlockSpec((1, 1, DIM),
                             lambda i, te_r: (jnp.clip(te_r[i], 0, ES - 1), 0, 0)),
            ],
            out_specs=pl.BlockSpec((TILE, DIM), lambda i, te_r: (i, 0)),
        ),
        out_shape=jax.ShapeDtypeStruct((PADT, DIM), jnp.float32),
    )
    ysp = spec(te, xg,
               spec_W1, spec_b1.reshape(ES, 1, HID),
               spec_W2, spec_b2.reshape(ES, 1, DIM))

    sc_combine = pl.kernel(
        _sc_combine_body,
        out_type=jax.ShapeDtypeStruct((T, DIM), jnp.float32),
        mesh=mesh,
        scratch_types=[
            pltpu.VMEM((CT, DIM), jnp.float32),
            pltpu.VMEM((CT, DIM), jnp.float32),
            pltpu.VMEM((CT, DIM), jnp.float32),
            pltpu.VMEM((CT, DIM), jnp.float32),
            pltpu.VMEM((CT, DIM), jnp.float32),
            pltpu.VMEM((CT, DIM), jnp.float32),
            pltpu.VMEM((TPW,), jnp.int32),
            pltpu.VMEM((TPW,), jnp.int32),
            pltpu.VMEM((TPW, L), jnp.float32),
            pltpu.VMEM((TPW, L), jnp.float32),
            pltpu.SemaphoreType.DMA,
            pltpu.SemaphoreType.DMA,
            pltpu.SemaphoreType.DMA,
            pltpu.SemaphoreType.DMA,
        ],
    )
    out = sc_combine(ysp, sh_out, s0, s1, w0e, w1e)

    return (out.reshape(B, T, DIM), sl.reshape(B, T, ES), sp.reshape(B, T, ES),
            ti.reshape(B, T, K), tp.reshape(B, T, K), hl.reshape(B, T, EH),
            hp.reshape(B, T, EH))
